# Initial kernel scaffold; baseline (speedup 1.0000x reference)
import jax
import jax.numpy as jnp
from jax.experimental import pallas as pl


def kernel(users, pos, neg, edge_index, embedding_weight):
    raise NotImplementedError("write your pallas kernel here")


# speedup vs baseline: 40.0217x; 40.0217x over previous
import functools

import jax
import jax.numpy as jnp
from jax import lax
from jax.experimental import pallas as pl
from jax.experimental.pallas import tpu as pltpu
from jax.experimental.pallas import tpu_sc as plsc

NUM_LAYER = 3
REG_WEIGHT = 1e-4

_NC = 2
_NS = 16
_CB = 128
_NBUF = 4
_HD = 16


def _sc_mesh():
    return plsc.VectorSubcoreMesh(core_axis_name="c", subcore_axis_name="s")


@functools.lru_cache(maxsize=None)
def _make_prop(np_rows: int, chunks_pt: int):
    assert np_rows % _NS == 0 and chunks_pt % (2 * _NBUF) == 0
    rpt = np_rows // _NS
    r2 = chunks_pt // (2 * _NBUF)

    def kern(gl, gr, srcs, dsts, zeros_h, outl, outr,
             isrc, idst, rows_b, sacc, sis0, sid0, sis1, sid1, *gsem):
        c = lax.axis_index("c")
        s = lax.axis_index("s")

        pltpu.sync_copy(zeros_h.at[pl.ds(s * rpt, rpt)],
                        sacc.at[pl.ds(s * rpt, rpt)])
        plsc.subcore_barrier()

        base = s * chunks_pt

        def run(g, out):
            def fire_idx(rnd, p, ss, sd):
                off = base + rnd * _NBUF
                pltpu.async_copy(srcs.at[pl.ds(off, _NBUF)], isrc.at[p], ss)
                pltpu.async_copy(dsts.at[pl.ds(off, _NBUF)], idst.at[p], sd)

            def wait_idx(p, ss, sd):
                pltpu.make_async_copy(srcs.at[pl.ds(0, _NBUF)],
                                      isrc.at[p], ss).wait()
                pltpu.make_async_copy(dsts.at[pl.ds(0, _NBUF)],
                                      idst.at[p], sd).wait()

            def fire_gathers(p):
                for b in range(_NBUF):
                    k = p * _NBUF + b
                    pltpu.async_copy(g.at[isrc.at[p, b]], rows_b.at[k],
                                     gsem[k])

            def drain_half(p):
                for b in range(_NBUF):
                    k = p * _NBUF + b
                    pltpu.make_async_copy(g.at[isrc.at[p, b]], rows_b.at[k],
                                          gsem[k]).wait()
                    pltpu.sync_copy(rows_b.at[k], sacc.at[idst.at[p, b]],
                                    add=True)

            fire_idx(0, 0, sis0, sid0)
            wait_idx(0, sis0, sid0)
            fire_gathers(0)
            fire_idx(1, 1, sis1, sid1)

            def loop_body(i, carry):
                wait_idx(1, sis1, sid1)
                fire_gathers(1)
                drain_half(0)

                @pl.when(i < r2 - 1)
                def _():
                    fire_idx(2 * i + 2, 0, sis0, sid0)
                    wait_idx(0, sis0, sid0)
                    fire_gathers(0)

                drain_half(1)

                @pl.when(i < r2 - 1)
                def _():
                    fire_idx(2 * i + 3, 1, sis1, sid1)

                return carry

            lax.fori_loop(0, r2, loop_body, 0)
            plsc.subcore_barrier()
            pltpu.sync_copy(sacc.at[pl.ds(s * rpt, rpt)],
                            out.at[pl.ds(s * rpt, rpt)])

        @pl.when(c == 0)
        def _():
            run(gl, outl)

        @pl.when(c == 1)
        def _():
            run(gr, outr)

    half = jax.ShapeDtypeStruct((np_rows, _HD), jnp.float32)
    scratch = [
        pltpu.VMEM((2, _NBUF, _CB), jnp.int32),
        pltpu.VMEM((2, _NBUF, _CB), jnp.int32),
        pltpu.VMEM((2 * _NBUF, _CB, _HD), jnp.float32),
        pltpu.VMEM_SHARED((np_rows, _HD), jnp.float32),
    ] + [pltpu.SemaphoreType.DMA] * (4 + 2 * _NBUF)
    return pl.kernel(kern, out_type=(half, half), mesh=_sc_mesh(),
                     scratch_types=scratch,
                     compiler_params=pltpu.CompilerParams(
                         use_tc_tiling_on_sc=False))


@functools.lru_cache(maxsize=None)
def _make_deg(np_rows: int, chunks_pw: int):
    assert np_rows % _NS == 0 and chunks_pw % (2 * _NBUF) == 0
    rpt = np_rows // _NS
    r2 = chunks_pw // (2 * _NBUF)

    def kern(dsts, zeros_h, ones_h, out0, out1,
             idst, vones, sacc, sid0, sid1):
        c = lax.axis_index("c")
        s = lax.axis_index("s")
        pltpu.sync_copy(zeros_h.at[pl.ds(s * rpt, rpt)],
                        sacc.at[pl.ds(s * rpt, rpt)])
        pltpu.sync_copy(ones_h.at[pl.ds(0, _CB)], vones)
        plsc.subcore_barrier()

        w = s * _NC + c
        base = w * chunks_pw

        def fire_idx(rnd, p, sd):
            pltpu.async_copy(dsts.at[pl.ds(base + rnd * _NBUF, _NBUF)],
                             idst.at[p], sd)

        def wait_idx(p, sd):
            pltpu.make_async_copy(dsts.at[pl.ds(0, _NBUF)],
                                  idst.at[p], sd).wait()

        def scatter_half(p):
            for b in range(_NBUF):
                pltpu.sync_copy(vones, sacc.at[idst.at[p, b]], add=True)

        fire_idx(0, 0, sid0)

        def loop_body(i, carry):
            wait_idx(0, sid0)
            fire_idx(2 * i + 1, 1, sid1)
            scatter_half(0)

            @pl.when(i < r2 - 1)
            def _():
                fire_idx(2 * i + 2, 0, sid0)

            wait_idx(1, sid1)
            scatter_half(1)
            return carry

        lax.fori_loop(0, r2, loop_body, 0)
        plsc.subcore_barrier()

        @pl.when(c == 0)
        def _():
            pltpu.sync_copy(sacc.at[pl.ds(s * rpt, rpt)],
                            out0.at[pl.ds(s * rpt, rpt)])

        @pl.when(c == 1)
        def _():
            pltpu.sync_copy(sacc.at[pl.ds(s * rpt, rpt)],
                            out1.at[pl.ds(s * rpt, rpt)])

    half = jax.ShapeDtypeStruct((np_rows, _HD), jnp.float32)
    scratch = [
        pltpu.VMEM((2, _NBUF, _CB), jnp.int32),
        pltpu.VMEM((_CB, _HD), jnp.float32),
        pltpu.VMEM_SHARED((np_rows, _HD), jnp.float32),
        pltpu.SemaphoreType.DMA, pltpu.SemaphoreType.DMA,
    ]
    return pl.kernel(kern, out_type=(half, half), mesh=_sc_mesh(),
                     scratch_types=scratch,
                     compiler_params=pltpu.CompilerParams(
                         use_tc_tiling_on_sc=False))


@functools.lru_cache(maxsize=None)
def _make_lookup(np_rows: int, n_idx: int):
    nw = _NC * _NS
    assert n_idx % (_CB * nw) == 0
    cpw = n_idx // _CB // nw

    def kern(accl, accr, el, er, idx2d, oal, oar, oel, oer, iv, rv, sem):
        c = lax.axis_index("c")
        s = lax.axis_index("s")
        w = s * _NC + c
        pltpu.sync_copy(idx2d.at[pl.ds(w * cpw, cpw)], iv)
        tabs = (accl, accr, el, er)
        outs = (oal, oar, oel, oer)
        descs = []
        for j in range(cpw):
            for t in range(4):
                descs.append(pltpu.async_copy(tabs[t].at[iv.at[j]],
                                              rv.at[j * 4 + t], sem))
        for dsc in descs:
            dsc.wait()
        for j in range(cpw):
            for t in range(4):
                pltpu.sync_copy(rv.at[j * 4 + t],
                                outs[t].at[pl.ds((w * cpw + j) * _CB, _CB)])

    out = jax.ShapeDtypeStruct((n_idx, _HD), jnp.float32)
    scratch = [
        pltpu.VMEM((cpw, _CB), jnp.int32),
        pltpu.VMEM((cpw * 4, _CB, _HD), jnp.float32),
        pltpu.SemaphoreType.DMA,
    ]
    return pl.kernel(kern, out_type=(out,) * 4, mesh=_sc_mesh(),
                     scratch_types=scratch,
                     compiler_params=pltpu.CompilerParams(
                         use_tc_tiling_on_sc=False))


def _init_body(deg0r, deg1r, elr, err, dr, glr, grr):
    deg = deg0r[...] + deg1r[...]
    d = jnp.where(deg > 0.5, lax.rsqrt(deg), 0.0)
    dr[...] = d
    glr[...] = d * elr[...]
    grr[...] = d * err[...]


def _scale_body(slr, srr, dr, alr, arr, glr, grr, oalr, oarr):
    d = dr[...]
    dd = d * d
    sl = slr[...]
    sr = srr[...]
    glr[...] = dd * sl
    grr[...] = dd * sr
    oalr[...] = alr[...] + d * sl
    oarr[...] = arr[...] + d * sr


def _scale_last_body(slr, srr, dr, alr, arr, oalr, oarr):
    d = dr[...]
    oalr[...] = alr[...] + d * slr[...]
    oarr[...] = arr[...] + d * srr[...]


@functools.lru_cache(maxsize=None)
def _make_elemwise(np_rows: int, n_in: int, n_out: int, which: str):
    body = {"init": _init_body, "scale": _scale_body,
            "last": _scale_last_body}[which]
    r = np_rows * _HD // 128
    bs = r // 16
    assert r % 16 == 0
    spec = pl.BlockSpec((bs, 128), lambda i: (i, 0))
    return pl.pallas_call(
        body, grid=(16,),
        in_specs=[spec] * n_in, out_specs=[spec] * n_out,
        out_shape=[jax.ShapeDtypeStruct((r, 128), jnp.float32)] * n_out)


@functools.lru_cache(maxsize=None)
def _make_loss(n_idx: int, batch: int):
    def body(alr, arr, elr, err, l_ref, le_ref, rg_ref):
        al = alr[...] * 0.25
        ar = arr[...] * 0.25
        ual, pal, nal = al[:batch], al[batch:2 * batch], al[2 * batch:]
        uar, par, nar = ar[:batch], ar[batch:2 * batch], ar[2 * batch:]
        pos = jnp.sum(ual * pal, axis=1) + jnp.sum(uar * par, axis=1)
        neg = jnp.sum(ual * nal, axis=1) + jnp.sum(uar * nar, axis=1)
        x = neg - pos
        sp = jnp.maximum(x, 0.0) + jnp.log(1.0 + jnp.exp(-jnp.abs(x)))
        le = jnp.mean(sp)
        el = elr[...]
        er = err[...]
        rg = (0.5 * (jnp.sum(el * el) + jnp.sum(er * er)) / batch) * REG_WEIGHT
        le_ref[...] = le.reshape(1, 1)
        rg_ref[...] = rg.reshape(1, 1)
        l_ref[...] = (le + rg).reshape(1, 1)

    return pl.pallas_call(
        body,
        out_shape=[jax.ShapeDtypeStruct((1, 1), jnp.float32)] * 3)


def kernel(users, pos, neg, edge_index, embedding_weight):
    n, demb = embedding_weight.shape
    e = edge_index.shape[1]
    batch = users.shape[0]
    assert demb == 2 * _HD

    np_rows = ((n + 1024) // 1024) * 1024
    chunks_pt = -(-e // (_NS * _CB))
    chunks_pt += (-chunks_pt) % (4 * _NBUF)
    e_pad = chunks_pt * _NS * _CB

    src = edge_index[0].astype(jnp.int32)
    dst = edge_index[1].astype(jnp.int32)
    pad = jnp.full((e_pad - e,), n, jnp.int32)
    srcs2d = jnp.concatenate([src, pad]).reshape(-1, _CB)
    dsts2d = jnp.concatenate([dst, pad]).reshape(-1, _CB)

    epad = jnp.pad(embedding_weight.astype(jnp.float32),
                   ((0, np_rows - n), (0, 0)))
    el = epad[:, :_HD]
    er = epad[:, _HD:]
    zeros_h = jnp.zeros((np_rows, _HD), jnp.float32)
    ones_h = jnp.ones((_CB, _HD), jnp.float32)

    prop = _make_prop(np_rows, chunks_pt)
    r = np_rows * _HD // 128

    def r128(a):
        return a.reshape(r, 128)

    def unr(a):
        return a.reshape(np_rows, _HD)

    chunks_pw = e_pad // (_NC * _NS * _CB)
    deg0, deg1 = _make_deg(np_rows, chunks_pw)(dsts2d, zeros_h, ones_h)

    d_r, gl_r, gr_r = _make_elemwise(np_rows, 4, 3, "init")(
        r128(deg0), r128(deg1), r128(el), r128(er))
    accl_r, accr_r = r128(el), r128(er)

    for layer in range(NUM_LAYER):
        sl, sr = prop(unr(gl_r), unr(gr_r), srcs2d, dsts2d, zeros_h)
        if layer < NUM_LAYER - 1:
            gl_r, gr_r, accl_r, accr_r = _make_elemwise(
                np_rows, 5, 4, "scale")(r128(sl), r128(sr), d_r,
                                        accl_r, accr_r)
        else:
            accl_r, accr_r = _make_elemwise(
                np_rows, 5, 2, "last")(r128(sl), r128(sr), d_r,
                                       accl_r, accr_r)

    n_idx = 3 * batch
    idx2d = jnp.concatenate([users, pos, neg]).astype(jnp.int32).reshape(
        -1, _CB)
    gal, gar, gel, ger = _make_lookup(np_rows, n_idx)(
        unr(accl_r), unr(accr_r), el, er, idx2d)

    l, le, rg = _make_loss(n_idx, batch)(gal, gar, gel, ger)
    return (l[0, 0], le[0, 0], rg[0, 0])
